# baseline (speedup 1.0000x reference)
import jax
import jax.numpy as jnp
from jax import lax
from jax.experimental import pallas as pl

N = 2048
OUT_C = 16
NEG_SLOPE = 0.2


def _gat_kernel(xc_ref, xr_ref, w_ref, as_ref, ad_ref, b_ref, out_ref):
    f32 = jnp.float32
    bf16 = jnp.bfloat16
    bbar = jnp.mean(b_ref[0, :])

    xrow = xr_ref[:, :]
    xcol = xc_ref[:, :]
    u_row = jnp.zeros((1, N), f32)
    u_col = jnp.zeros((N, 1), f32)
    c_row = jnp.zeros((1, N), f32)
    hbar = jnp.zeros((1, N), f32)
    for ch in range(OUT_C):
        wc = w_ref[0, ch]
        asc = as_ref[0, ch].astype(bf16).astype(f32)
        adc = ad_ref[0, ch].astype(bf16).astype(f32)
        h_c = xrow * wc
        hb_c = h_c.astype(bf16).astype(f32)
        u_row = u_row + hb_c * asc
        c_row = c_row + hb_c * adc
        hbar = hbar + h_c
        u_col = u_col + (xcol * wc).astype(bf16).astype(f32) * asc
    hbar = hbar * (1.0 / OUT_C)

    umax = jnp.max(u_row)
    q = jnp.exp(u_row - umax)
    r = jnp.exp(NEG_SLOPE * (u_row - umax))
    v4 = jnp.concatenate([q, q * hbar, r, r * hbar], axis=0)
    rtot = jnp.sum(v4[2:3, :])
    rxtot = jnp.sum(v4[3:4, :])

    thr = -c_row
    mask_f = jnp.where(u_col > thr, 1.0, 0.0)
    mask = mask_f.astype(bf16)
    v_hi = v4.astype(bf16)
    v_lo = (v4 - v_hi.astype(f32)).astype(bf16)
    dn = (((1,), (0,)), ((), ()))
    sums = (lax.dot_general(v_hi, mask, dn, preferred_element_type=f32)
            + lax.dot_general(v_lo, mask, dn, preferred_element_type=f32))
    a = sums[0:1, :]
    ax = sums[1:2, :]
    bsum = rtot - sums[2:3, :]
    bxsum = rxtot - sums[3:4, :]

    g = umax + c_row
    m = jnp.where(g > 0, g, NEG_SLOPE * g)
    f1 = jnp.exp(g - m)
    f2 = jnp.exp(NEG_SLOPE * g - m)
    denom = f1 * a + f2 * bsum + 1e-16
    numer = f1 * ax + f2 * bxsum
    out_ref[:, :] = numer / denom + bbar


def kernel(x, W, att_src, att_dst, bias):
    a, b, n, d = x.shape
    xf = x.reshape(n, 1)
    xr = x.reshape(1, n)
    w2 = W.reshape(1, -1)
    as2 = att_src.reshape(1, -1)
    ad2 = att_dst.reshape(1, -1)
    b2 = bias.reshape(1, -1)

    out = pl.pallas_call(
        _gat_kernel,
        out_shape=jax.ShapeDtypeStruct((1, n), jnp.float32),
    )(xf, xr, w2, as2, ad2, b2)

    return out.reshape(n, a, b, d).transpose(1, 2, 0, 3)


# speedup vs baseline: 2.0389x; 2.0389x over previous
import jax
import jax.numpy as jnp
from jax import lax
from jax.experimental import pallas as pl

N = 2048
OUT_C = 16
NEG_SLOPE = 0.2


def _gat_kernel(xr_ref, w_ref, as_ref, ad_ref, b_ref, out_ref):
    f32 = jnp.float32
    bf16 = jnp.bfloat16
    bbar = jnp.mean(b_ref[0, :])

    xrow = xr_ref[:, :]
    u_row = jnp.zeros((1, N), f32)
    c_row = jnp.zeros((1, N), f32)
    hbar = jnp.zeros((1, N), f32)
    for ch in range(OUT_C):
        wc = w_ref[0, ch]
        asc = as_ref[0, ch].astype(bf16).astype(f32)
        adc = ad_ref[0, ch].astype(bf16).astype(f32)
        h_c = xrow * wc
        hb_c = h_c.astype(bf16).astype(f32)
        u_row = u_row + hb_c * asc
        c_row = c_row + hb_c * adc
        hbar = hbar + h_c
    hbar = hbar * (1.0 / OUT_C)
    u_col = jnp.transpose(u_row, (1, 0))

    umax = jnp.max(u_row)
    q = jnp.exp(u_row - umax)
    r = jnp.exp(NEG_SLOPE * (u_row - umax))
    v4 = jnp.concatenate([q, q * hbar, r, r * hbar], axis=0)
    rtot = jnp.sum(v4[2:3, :])
    rxtot = jnp.sum(v4[3:4, :])

    thr = -c_row
    mask_f = jnp.where(u_col > thr, 1.0, 0.0)
    mask = mask_f.astype(bf16)
    v_hi = v4.astype(bf16)
    v_lo = (v4 - v_hi.astype(f32)).astype(bf16)
    dn = (((1,), (0,)), ((), ()))
    sums = (lax.dot_general(v_hi, mask, dn, preferred_element_type=f32)
            + lax.dot_general(v_lo, mask, dn, preferred_element_type=f32))
    a = sums[0:1, :]
    ax = sums[1:2, :]
    bsum = rtot - sums[2:3, :]
    bxsum = rxtot - sums[3:4, :]

    g = umax + c_row
    m = jnp.where(g > 0, g, NEG_SLOPE * g)
    f1 = jnp.exp(g - m)
    f2 = jnp.exp(NEG_SLOPE * g - m)
    denom = f1 * a + f2 * bsum + 1e-16
    numer = f1 * ax + f2 * bxsum
    out_ref[:, :] = numer / denom + bbar


def kernel(x, W, att_src, att_dst, bias):
    a, b, n, d = x.shape
    xr = x.reshape(1, n)
    w2 = W.reshape(1, -1)
    as2 = att_src.reshape(1, -1)
    ad2 = att_dst.reshape(1, -1)
    b2 = bias.reshape(1, -1)

    out = pl.pallas_call(
        _gat_kernel,
        out_shape=jax.ShapeDtypeStruct((1, n), jnp.float32),
    )(xr, w2, as2, ad2, b2)

    return out.reshape(n, a, b, d).transpose(1, 2, 0, 3)
